# baseline (speedup 1.0000x reference)
import functools
import jax
import jax.numpy as jnp
from jax import lax
from jax.experimental import pallas as pl
from jax.experimental.pallas import tpu as pltpu
from jax.experimental.pallas import tpu_sc as plsc

D_MODEL_ = 32
NUM_EXPERTS_ = 128
N_TOKENS_ = 8192
D_FF_ = 4 * D_MODEL_
NW_ = 32
EPW_ = NUM_EXPERTS_ // NW_
L_ = 16
NT_ = NUM_EXPERTS_ // L_


def _perm(v, idx):
    dnums = lax.GatherDimensionNumbers(
        offset_dims=(), collapsed_slice_dims=(0,), start_index_map=(0,))
    return lax.gather(v, idx[:, None], dnums, (1,),
                      mode=lax.GatherScatterMode.PROMISE_IN_BOUNDS)


def _bcast(v, j):
    return _perm(v, jnp.full((L_,), j, jnp.int32))


def _lane_allreduce(v, op):
    idx = lax.iota(jnp.int32, L_)
    for shift in (8, 4, 2, 1):
        v = op(v, _perm(v, jnp.bitwise_xor(idx, shift)))
    return v


def _sc_combine_kernel(x_hbm, gwt_hbm, gb_hbm, ewt_hbm, mparts_hbm, w_hbm,
                       xrow_v, gwt_v, gb_v, w_v, ewt_v, pm_v):
    wid = lax.axis_index("s") * 2 + lax.axis_index("c")
    base = wid * EPW_
    pltpu.sync_copy(x_hbm.at[pl.ds(base, EPW_)], xrow_v)
    pltpu.sync_copy(gwt_hbm, gwt_v)
    pltpu.sync_copy(gb_hbm, gb_v)
    pltpu.sync_copy(ewt_hbm.at[pl.ds(base, EPW_)], ewt_v)

    for e in range(EPW_):
        xv = [xrow_v[e, pl.ds(L_ * g, L_)] for g in range(D_MODEL_ // L_)]
        xs = [_bcast(xv[d // L_], d % L_) for d in range(D_MODEL_)]
        es = []
        for t in range(NT_):
            acc = gb_v[pl.ds(L_ * t, L_)]
            for d in range(D_MODEL_):
                acc = acc + xs[d] * gwt_v[d, pl.ds(L_ * t, L_)]
            es.append(acc)
        vmax = _lane_allreduce(functools.reduce(jnp.maximum, es), jnp.maximum)
        es = [jnp.exp(v - vmax) for v in es]
        vsum = _lane_allreduce(functools.reduce(jnp.add, es), jnp.add)
        rinv = 1.0 / vsum
        for t in range(NT_):
            w_v[e, pl.ds(L_ * t, L_)] = es[t] * rinv
    pltpu.sync_copy(w_v, w_hbm.at[pl.ds(base, EPW_)])

    for t in range(NT_):
        wv = [w_v[e, pl.ds(L_ * t, L_)] for e in range(EPW_)]
        for d in range(D_MODEL_):
            acc = wv[0] * ewt_v[0, d, pl.ds(L_ * t, L_)]
            for e in range(1, EPW_):
                acc = acc + wv[e] * ewt_v[e, d, pl.ds(L_ * t, L_)]
            pm_v[d, pl.ds(L_ * t, L_)] = acc
    pltpu.sync_copy(pm_v, mparts_hbm.at[wid])


def _tc_matmul_kernel(x_ref, mparts_ref, w_ref, eb_ref, o_ref):
    mt = jnp.sum(mparts_ref[...], axis=0)
    b2 = jnp.sum(w_ref[...] * eb_ref[...], axis=0)
    o_ref[...] = jnp.dot(x_ref[...], mt,
                         preferred_element_type=jnp.float32) + b2[None, :]


def kernel(x, gate_w, gate_b, expert_w, expert_b):
    ewt = jnp.transpose(expert_w, (0, 2, 1))
    mesh = plsc.VectorSubcoreMesh(core_axis_name="c", subcore_axis_name="s")
    sc = functools.partial(
        pl.kernel, mesh=mesh,
        out_type=(
            jax.ShapeDtypeStruct((NW_, D_MODEL_, NUM_EXPERTS_), jnp.float32),
            jax.ShapeDtypeStruct((NUM_EXPERTS_, NUM_EXPERTS_), jnp.float32),
        ),
        scratch_types=[
            pltpu.VMEM((EPW_, D_MODEL_), jnp.float32),
            pltpu.VMEM((D_MODEL_, NUM_EXPERTS_), jnp.float32),
            pltpu.VMEM((NUM_EXPERTS_,), jnp.float32),
            pltpu.VMEM((EPW_, NUM_EXPERTS_), jnp.float32),
            pltpu.VMEM((EPW_, D_MODEL_, NUM_EXPERTS_), jnp.float32),
            pltpu.VMEM((D_MODEL_, NUM_EXPERTS_), jnp.float32),
        ],
    )(_sc_combine_kernel)
    mparts, w = sc(x[:NUM_EXPERTS_], gate_w.T, gate_b, ewt)
    return pl.pallas_call(
        _tc_matmul_kernel,
        out_shape=jax.ShapeDtypeStruct((N_TOKENS_, NUM_EXPERTS_), jnp.float32),
    )(x, mparts, w, expert_b)


# speedup vs baseline: 3.6490x; 3.6490x over previous
import jax
import jax.numpy as jnp
from jax.experimental import pallas as pl

D_MODEL_ = 32
NUM_EXPERTS_ = 128
N_TOKENS_ = 8192
D_FF_ = 4 * D_MODEL_


def _moe_kernel(x_ref, gw_ref, gb_ref, ewt_ref, eb_ref, o_ref):
    xg = x_ref[:NUM_EXPERTS_, :]
    logits = jnp.dot(xg, gw_ref[...].T,
                     preferred_element_type=jnp.float32) + gb_ref[...]
    w = jax.nn.softmax(logits, axis=-1)
    mt = jnp.sum(ewt_ref[...] * w[:, None, :], axis=0)
    b2 = jnp.sum(w * eb_ref[...], axis=0)
    o_ref[...] = jnp.dot(x_ref[...], mt,
                         preferred_element_type=jnp.float32) + b2[None, :]


def kernel(x, gate_w, gate_b, expert_w, expert_b):
    ewt = jnp.transpose(expert_w, (0, 2, 1))
    gb = gate_b.reshape(1, NUM_EXPERTS_)
    return pl.pallas_call(
        _moe_kernel,
        out_shape=jax.ShapeDtypeStruct((N_TOKENS_, NUM_EXPERTS_), jnp.float32),
    )(x, gate_w, gb, ewt, expert_b)
